# baseline (speedup 1.0000x reference)
import functools
import jax
import jax.numpy as jnp
from jax import lax
from jax.experimental import pallas as pl
from jax.experimental.pallas import tpu as pltpu
from jax.experimental.pallas import tpu_sc as plsc

N_CHANNELS = [16, 32, 64, 128]
PYRAMID = [2048, 512, 128]
K_NN = 16
_SC_CORES = 2
_SC_SUBCORES = 16
_NW = _SC_CORES * _SC_SUBCORES


def _mlp_chain_body(nlayers, *refs):
    x_ref = refs[0]
    o_ref = refs[-1]
    h = x_ref[0]
    for i in range(nlayers):
        w = refs[1 + 2 * i][...]
        b = refs[2 + 2 * i][...]
        h = jnp.dot(h, w, preferred_element_type=jnp.float32) + b
        h = jnp.where(h >= 0, h, 0.1 * h)
    o_ref[0] = h


def _mlp_chain(xT, layers, TN=2048):
    B, N, C = xT.shape
    TN = min(TN, N)
    Cout = layers[-1][0].shape[1]
    in_specs = [pl.BlockSpec((1, TN, C), lambda b, n: (b, n, 0))]
    args = [xT]
    for (W, bb) in layers:
        ci, co = W.shape
        in_specs.append(pl.BlockSpec((ci, co), lambda b, n: (0, 0)))
        in_specs.append(pl.BlockSpec((1, co), lambda b, n: (0, 0)))
        args.append(W)
        args.append(bb)
    return pl.pallas_call(
        functools.partial(_mlp_chain_body, len(layers)),
        grid=(B, N // TN),
        in_specs=in_specs,
        out_specs=pl.BlockSpec((1, TN, Cout), lambda b, n: (b, n, 0)),
        out_shape=jax.ShapeDtypeStruct((B, N, Cout), jnp.float32),
    )(*args)


def _pre_body(nlayers, *refs):
    x_ref, xyz_ref = refs[0], refs[1]
    o_ref = refs[-1]
    wf_ref, wx_ref = refs[-3], refs[-2]
    h = x_ref[0]
    for i in range(nlayers):
        w = refs[2 + 2 * i][...]
        b = refs[3 + 2 * i][...]
        h = jnp.dot(h, w, preferred_element_type=jnp.float32) + b
        h = jnp.where(h >= 0, h, 0.1 * h)
    pre = (jnp.dot(h, wf_ref[...], preferred_element_type=jnp.float32)
           + jnp.dot(xyz_ref[0], wx_ref[...], preferred_element_type=jnp.float32))
    o_ref[0] = pre


def _pre_kernel(fT, xyz8T, layers, WfT, WxT, TN=2048):
    B, N, C = fT.shape
    TN = min(TN, N)
    Co = WfT.shape[1]
    in_specs = [pl.BlockSpec((1, TN, C), lambda b, n: (b, n, 0)),
                pl.BlockSpec((1, TN, 8), lambda b, n: (b, n, 0))]
    args = [fT, xyz8T]
    for (W, bb) in layers:
        ci, co = W.shape
        in_specs.append(pl.BlockSpec((ci, co), lambda b, n: (0, 0)))
        in_specs.append(pl.BlockSpec((1, co), lambda b, n: (0, 0)))
        args.append(W)
        args.append(bb)
    in_specs.append(pl.BlockSpec(WfT.shape, lambda b, n: (0, 0)))
    in_specs.append(pl.BlockSpec(WxT.shape, lambda b, n: (0, 0)))
    args += [WfT, WxT]
    return pl.pallas_call(
        functools.partial(_pre_body, len(layers)),
        grid=(B, N // TN),
        in_specs=in_specs,
        out_specs=pl.BlockSpec((1, TN, Co), lambda b, n: (b, n, 0)),
        out_shape=jax.ShapeDtypeStruct((B, N, Co), jnp.float32),
    )(*args)


def _knn_body(N, TM, xyz_ref, new_ref, wx_ref, b_ref, idx_ref, q_ref):
    b = pl.program_id(0)
    xyz = xyz_ref[0]
    new = new_ref[0]
    nx = ((xyz[0:1] * xyz[0:1] + xyz[1:2] * xyz[1:2])
          + xyz[2:3] * xyz[2:3])
    nq = ((new[:, 0:1] * new[:, 0:1] + new[:, 1:2] * new[:, 1:2])
          + new[:, 2:3] * new[:, 2:3])
    G = jnp.dot(new, xyz, preferred_element_type=jnp.float32)
    D = (nq + nx) - 2.0 * G
    S = 8
    W = N // S
    lane_w = lax.broadcasted_iota(jnp.int32, (TM, W), 1).astype(jnp.float32)
    vals = [D[:, t * W:(t + 1) * W] for t in range(S)]
    pays = [lane_w + jnp.float32(t * W) for t in range(S)]
    NET8 = [(0, 1), (2, 3), (0, 2), (1, 3), (1, 2),
            (4, 5), (6, 7), (4, 6), (5, 7), (5, 6),
            (0, 4), (1, 5), (2, 6), (3, 7), (2, 4), (3, 5),
            (1, 2), (3, 4), (5, 6)]
    for (i, j) in NET8:
        a, bb, pa, pb = vals[i], vals[j], pays[i], pays[j]
        sw = a <= bb
        vals[i] = jnp.minimum(a, bb)
        vals[j] = jnp.maximum(a, bb)
        pays[i] = jnp.where(sw, pa, pb)
        pays[j] = jnp.where(sw, pb, pa)
    BIGF = jnp.float32(3e38)
    cols = []
    for r in range(K_NN):
        v = jnp.min(vals[0], axis=1, keepdims=True)
        cand = jnp.where(vals[0] <= v, pays[0], BIGF)
        im = jnp.min(cand, axis=1, keepdims=True)
        cols.append(im)
        if r == K_NN - 1:
            break
        pop = cand == im
        depth = min(S - 1, K_NN - 1 - r)
        for t in range(depth):
            vals[t] = jnp.where(pop, vals[t + 1], vals[t])
            pays[t] = jnp.where(pop, pays[t + 1], pays[t])
        if depth == S - 1:
            vals[S - 1] = jnp.where(pop, BIGF, vals[S - 1])
    idx = jnp.concatenate(cols, axis=1).astype(jnp.int32)
    idx_ref[0] = idx + b * N
    q_ref[0] = (jnp.dot(new, wx_ref[...], preferred_element_type=jnp.float32)
                - b_ref[...])


def _knn_kernel(xyz8, new8T, WxT, convb, TM=256):
    B, _, N = xyz8.shape
    M = new8T.shape[1]
    Co = WxT.shape[1]
    TM = min(TM, M)
    return pl.pallas_call(
        functools.partial(_knn_body, N, TM),
        grid=(B, M // TM),
        in_specs=[
            pl.BlockSpec((1, 8, N), lambda b, m: (b, 0, 0)),
            pl.BlockSpec((1, TM, 8), lambda b, m: (b, m, 0)),
            pl.BlockSpec((8, Co), lambda b, m: (0, 0)),
            pl.BlockSpec((1, Co), lambda b, m: (0, 0)),
        ],
        out_specs=[
            pl.BlockSpec((1, TM, K_NN), lambda b, m: (b, m, 0)),
            pl.BlockSpec((1, TM, Co), lambda b, m: (b, m, 0)),
        ],
        out_shape=[
            jax.ShapeDtypeStruct((B, M, K_NN), jnp.int32),
            jax.ShapeDtypeStruct((B, M, Co), jnp.float32),
        ],
    )(xyz8, new8T, WxT, convb)


def _sc_gather_max(table, idxflat, qflat):
    BN, TW = table.shape
    Co = qflat.shape[1]
    NQ = qflat.shape[0]
    qpw = NQ // _NW
    QB = min(16, max(8, qpw // 2))
    GR = QB * K_NN
    NG = GR // 128
    nchunks = qpw // QB
    npairs = nchunks // 2
    mesh = plsc.VectorSubcoreMesh(core_axis_name="c", subcore_axis_name="s")

    @functools.partial(
        pl.kernel, mesh=mesh,
        out_type=jax.ShapeDtypeStruct((NQ, Co), jnp.float32),
        scratch_types=[
            pltpu.VMEM((2, GR), jnp.int32),
            pltpu.VMEM((2, GR, TW), jnp.float32),
            pltpu.VMEM((2, QB, Co), jnp.float32),
            pltpu.VMEM((2, QB, Co), jnp.float32),
            pltpu.SemaphoreType.DMA,
            pltpu.SemaphoreType.DMA,
        ],
    )
    def k(table_hbm, idx_hbm, q_hbm, out_hbm, idx_v, rows_v, q_v, out_v,
          sem0, sem1):
        wid = lax.axis_index("s") * _SC_CORES + lax.axis_index("c")
        base_q = wid * qpw
        sems = (sem0, sem1)

        def pair(j2, carry):
            handles = []
            for par in range(2):
                q0 = base_q + (2 * j2 + par) * QB
                pltpu.sync_copy(idx_hbm.at[pl.ds(q0 * K_NN, GR)],
                                idx_v.at[par])
                for g in range(NG):
                    handles.append(pltpu.async_copy(
                        table_hbm.at[idx_v.at[par, pl.ds(g * 128, 128)]],
                        rows_v.at[par, pl.ds(g * 128, 128)], sems[par]))
                pltpu.sync_copy(q_hbm.at[pl.ds(q0, QB)], q_v.at[par])
            for par in range(2):
                q0 = base_q + (2 * j2 + par) * QB
                for g in range(NG):
                    handles[par * NG + g].wait()
                for q in range(QB):
                    for c in range(Co // 16):
                        sl = pl.ds(c * 16, 16)
                        acc = rows_v[par, q * K_NN, sl]
                        for kk in range(1, K_NN):
                            acc = jnp.maximum(acc, rows_v[par, q * K_NN + kk, sl])
                        h = acc - q_v[par, q, sl]
                        out_v[par, q, sl] = jnp.where(h >= 0, h, 0.1 * h)
                pltpu.sync_copy(out_v.at[par], out_hbm.at[pl.ds(q0, QB)])
            return carry

        lax.fori_loop(0, npairs, pair, 0)

    return k(table, idxflat, qflat)


def _pyramid(pcd):
    xyzs = [pcd]
    cur = pcd
    for n in PYRAMID:
        stride = cur.shape[2] // n
        cur = cur[:, :, ::stride][:, :, :n]
        xyzs.append(cur)
    return xyzs


def kernel(pcd, lvl0_W0, lvl0_b0, lvl0_W1, lvl0_b1, mlp0_W0, mlp0_b0, mlp0_W1, mlp0_b1, conv0_W, conv0_b, mlp1_W0, mlp1_b0, mlp1_W1, mlp1_b1, conv1_W, conv1_b, mlp2_W0, mlp2_b0, mlp2_W1, mlp2_b1, conv2_W, conv2_b):
    B, _, N0 = pcd.shape
    xyzs = _pyramid(pcd)
    Ns = [N0] + PYRAMID
    xyzT = [jnp.transpose(x, (0, 2, 1)) for x in xyzs]
    xyz8T = [jnp.pad(x, ((0, 0), (0, 0), (0, 5))) for x in xyzT]
    xyz8 = [jnp.transpose(x, (0, 2, 1)) for x in xyz8T]

    feats0T = _mlp_chain(
        xyzT[0],
        [(lvl0_W0.T, lvl0_b0[None, :]), (lvl0_W1.T, lvl0_b1[None, :])])

    mlps = [((mlp0_W0, mlp0_b0), (mlp0_W1, mlp0_b1)),
            ((mlp1_W0, mlp1_b0), (mlp1_W1, mlp1_b1)),
            ((mlp2_W0, mlp2_b0), (mlp2_W1, mlp2_b1))]
    convs = [(conv0_W, conv0_b), (conv1_W, conv1_b), (conv2_W, conv2_b)]

    TW = 128
    knn = []
    for i in range(3):
        convW, convbias = convs[i]
        Co = convW.shape[0]
        WxT = jnp.pad(jnp.transpose(convW[:, :3], (1, 0)),
                      ((0, 5), (0, 0)))
        knn.append(_knn_kernel(xyz8[i], xyz8T[i + 1], WxT,
                               convbias[None, :]))

    fT = feats0T
    featsT = [feats0T]
    for i in range(3):
        convW, convbias = convs[i]
        Co = convW.shape[0]
        layers = [(W.T, bb[None, :]) for (W, bb) in mlps[i]]
        WfT = jnp.transpose(convW[:, 3:], (1, 0))
        WxT = jnp.pad(jnp.transpose(convW[:, :3], (1, 0)),
                      ((0, 5), (0, 0)))
        WfTp = jnp.pad(WfT, ((0, 0), (0, TW - Co)))
        WxTp = jnp.pad(WxT, ((0, 0), (0, TW - Co)))
        preT = _pre_kernel(fT, xyz8T[i], layers, WfTp, WxTp)
        idxf, qT = knn[i]
        M = Ns[i + 1]
        out = _sc_gather_max(
            preT.reshape(B * Ns[i], TW),
            idxf.reshape(B * M * K_NN),
            qT.reshape(B * M, Co))
        fT = out.reshape(B, M, Co)
        featsT.append(fT)

    feats = tuple(jnp.transpose(f, (0, 2, 1)) for f in featsT)
    return feats + tuple(xyzs)


# speedup vs baseline: 1.0148x; 1.0148x over previous
import functools
import jax
import jax.numpy as jnp
from jax import lax
from jax.experimental import pallas as pl
from jax.experimental.pallas import tpu as pltpu
from jax.experimental.pallas import tpu_sc as plsc

N_CHANNELS = [16, 32, 64, 128]
PYRAMID = [2048, 512, 128]
K_NN = 16
_SC_CORES = 2
_SC_SUBCORES = 16
_NW = _SC_CORES * _SC_SUBCORES


def _mlp_chain_body(nlayers, *refs):
    x_ref = refs[0]
    o_ref = refs[-1]
    h = x_ref[0]
    for i in range(nlayers):
        w = refs[1 + 2 * i][...]
        b = refs[2 + 2 * i][...]
        h = jnp.dot(h, w, preferred_element_type=jnp.float32) + b
        h = jnp.where(h >= 0, h, 0.1 * h)
    o_ref[0] = h


def _mlp_chain(xT, layers, TN=2048):
    B, N, C = xT.shape
    TN = min(TN, N)
    Cout = layers[-1][0].shape[1]
    in_specs = [pl.BlockSpec((1, TN, C), lambda b, n: (b, n, 0))]
    args = [xT]
    for (W, bb) in layers:
        ci, co = W.shape
        in_specs.append(pl.BlockSpec((ci, co), lambda b, n: (0, 0)))
        in_specs.append(pl.BlockSpec((1, co), lambda b, n: (0, 0)))
        args.append(W)
        args.append(bb)
    return pl.pallas_call(
        functools.partial(_mlp_chain_body, len(layers)),
        grid=(B, N // TN),
        in_specs=in_specs,
        out_specs=pl.BlockSpec((1, TN, Cout), lambda b, n: (b, n, 0)),
        out_shape=jax.ShapeDtypeStruct((B, N, Cout), jnp.float32),
    )(*args)


def _pre_body(nlayers, *refs):
    x_ref, xyz_ref = refs[0], refs[1]
    o_ref = refs[-1]
    wf_ref, wx_ref = refs[-3], refs[-2]
    h = x_ref[0]
    for i in range(nlayers):
        w = refs[2 + 2 * i][...]
        b = refs[3 + 2 * i][...]
        h = jnp.dot(h, w, preferred_element_type=jnp.float32) + b
        h = jnp.where(h >= 0, h, 0.1 * h)
    pre = (jnp.dot(h, wf_ref[...], preferred_element_type=jnp.float32)
           + jnp.dot(xyz_ref[0], wx_ref[...], preferred_element_type=jnp.float32))
    o_ref[0] = pre


def _pre_kernel(fT, xyz8T, layers, WfT, WxT, TN=2048):
    B, N, C = fT.shape
    TN = min(TN, N)
    Co = WfT.shape[1]
    in_specs = [pl.BlockSpec((1, TN, C), lambda b, n: (b, n, 0)),
                pl.BlockSpec((1, TN, 8), lambda b, n: (b, n, 0))]
    args = [fT, xyz8T]
    for (W, bb) in layers:
        ci, co = W.shape
        in_specs.append(pl.BlockSpec((ci, co), lambda b, n: (0, 0)))
        in_specs.append(pl.BlockSpec((1, co), lambda b, n: (0, 0)))
        args.append(W)
        args.append(bb)
    in_specs.append(pl.BlockSpec(WfT.shape, lambda b, n: (0, 0)))
    in_specs.append(pl.BlockSpec(WxT.shape, lambda b, n: (0, 0)))
    args += [WfT, WxT]
    return pl.pallas_call(
        functools.partial(_pre_body, len(layers)),
        grid=(B, N // TN),
        in_specs=in_specs,
        out_specs=pl.BlockSpec((1, TN, Co), lambda b, n: (b, n, 0)),
        out_shape=jax.ShapeDtypeStruct((B, N, Co), jnp.float32),
    )(*args)


def _knn_body(N, TM, xyz_ref, new_ref, wx_ref, b_ref, idx_ref, q_ref):
    b = pl.program_id(0)
    xyz = xyz_ref[0]
    new = new_ref[0]
    nx = ((xyz[0:1] * xyz[0:1] + xyz[1:2] * xyz[1:2])
          + xyz[2:3] * xyz[2:3])
    nq = ((new[:, 0:1] * new[:, 0:1] + new[:, 1:2] * new[:, 1:2])
          + new[:, 2:3] * new[:, 2:3])
    G = jnp.dot(new, xyz, preferred_element_type=jnp.float32)
    D = (nq + nx) - 2.0 * G
    S = 8
    W = N // S
    lane_w = lax.broadcasted_iota(jnp.int32, (TM, W), 1).astype(jnp.float32)
    vals = [D[:, t * W:(t + 1) * W] for t in range(S)]
    pays = [lane_w + jnp.float32(t * W) for t in range(S)]
    NET = [(0, 1), (2, 3), (0, 2), (1, 3), (1, 2),
           (4, 5), (6, 7), (4, 6), (5, 7), (5, 6),
           (0, 4), (1, 5), (2, 6), (3, 7), (2, 4), (3, 5),
           (1, 2), (3, 4), (5, 6)]
    for (i, j) in NET:
        a, bb, pa, pb = vals[i], vals[j], pays[i], pays[j]
        sw = a <= bb
        vals[i] = jnp.minimum(a, bb)
        vals[j] = jnp.maximum(a, bb)
        pays[i] = jnp.where(sw, pa, pb)
        pays[j] = jnp.where(sw, pb, pa)
    BIGF = jnp.float32(3e38)
    cols = []
    for r in range(K_NN):
        v = jnp.min(vals[0], axis=1, keepdims=True)
        cand = jnp.where(vals[0] <= v, pays[0], BIGF)
        im = jnp.min(cand, axis=1, keepdims=True)
        cols.append(im)
        if r == K_NN - 1:
            break
        pop = cand == im
        depth = min(S - 1, K_NN - 1 - r)
        for t in range(depth):
            vals[t] = jnp.where(pop, vals[t + 1], vals[t])
            pays[t] = jnp.where(pop, pays[t + 1], pays[t])
        if depth == S - 1:
            vals[S - 1] = jnp.where(pop, BIGF, vals[S - 1])
    idx = jnp.concatenate(cols, axis=1).astype(jnp.int32)
    idx_ref[0] = idx + b * N
    q_ref[0] = (jnp.dot(new, wx_ref[...], preferred_element_type=jnp.float32)
                - b_ref[...])


def _knn_kernel(xyz8, new8T, WxT, convb, TM=128):
    B, _, N = xyz8.shape
    M = new8T.shape[1]
    Co = WxT.shape[1]
    TM = min(TM, M)
    return pl.pallas_call(
        functools.partial(_knn_body, N, TM),
        grid=(B, M // TM),
        in_specs=[
            pl.BlockSpec((1, 8, N), lambda b, m: (b, 0, 0)),
            pl.BlockSpec((1, TM, 8), lambda b, m: (b, m, 0)),
            pl.BlockSpec((8, Co), lambda b, m: (0, 0)),
            pl.BlockSpec((1, Co), lambda b, m: (0, 0)),
        ],
        out_specs=[
            pl.BlockSpec((1, TM, K_NN), lambda b, m: (b, m, 0)),
            pl.BlockSpec((1, TM, Co), lambda b, m: (b, m, 0)),
        ],
        out_shape=[
            jax.ShapeDtypeStruct((B, M, K_NN), jnp.int32),
            jax.ShapeDtypeStruct((B, M, Co), jnp.float32),
        ],
    )(xyz8, new8T, WxT, convb)


def _sc_gather_max(table, idxflat, qflat):
    BN, TW = table.shape
    Co = qflat.shape[1]
    NQ = qflat.shape[0]
    qpw = NQ // _NW
    QB = min(16, max(8, qpw // 2))
    GR = QB * K_NN
    NG = GR // 128
    nchunks = qpw // QB
    npairs = nchunks // 2
    mesh = plsc.VectorSubcoreMesh(core_axis_name="c", subcore_axis_name="s")

    @functools.partial(
        pl.kernel, mesh=mesh,
        out_type=jax.ShapeDtypeStruct((NQ, Co), jnp.float32),
        scratch_types=[
            pltpu.VMEM((2, GR), jnp.int32),
            pltpu.VMEM((2, GR, TW), jnp.float32),
            pltpu.VMEM((2, QB, Co), jnp.float32),
            pltpu.VMEM((2, QB, Co), jnp.float32),
            pltpu.SemaphoreType.DMA,
            pltpu.SemaphoreType.DMA,
        ],
    )
    def k(table_hbm, idx_hbm, q_hbm, out_hbm, idx_v, rows_v, q_v, out_v,
          sem0, sem1):
        wid = lax.axis_index("s") * _SC_CORES + lax.axis_index("c")
        base_q = wid * qpw
        sems = (sem0, sem1)

        def pair(j2, carry):
            handles = []
            for par in range(2):
                q0 = base_q + (2 * j2 + par) * QB
                pltpu.sync_copy(idx_hbm.at[pl.ds(q0 * K_NN, GR)],
                                idx_v.at[par])
                for g in range(NG):
                    handles.append(pltpu.async_copy(
                        table_hbm.at[idx_v.at[par, pl.ds(g * 128, 128)]],
                        rows_v.at[par, pl.ds(g * 128, 128)], sems[par]))
                pltpu.sync_copy(q_hbm.at[pl.ds(q0, QB)], q_v.at[par])
            for par in range(2):
                q0 = base_q + (2 * j2 + par) * QB
                for g in range(NG):
                    handles[par * NG + g].wait()
                for q in range(QB):
                    for c in range(Co // 16):
                        sl = pl.ds(c * 16, 16)
                        acc = rows_v[par, q * K_NN, sl]
                        for kk in range(1, K_NN):
                            acc = jnp.maximum(acc, rows_v[par, q * K_NN + kk, sl])
                        h = acc - q_v[par, q, sl]
                        out_v[par, q, sl] = jnp.where(h >= 0, h, 0.1 * h)
                pltpu.sync_copy(out_v.at[par], out_hbm.at[pl.ds(q0, QB)])
            return carry

        lax.fori_loop(0, npairs, pair, 0)

    return k(table, idxflat, qflat)


def _pyramid(pcd):
    xyzs = [pcd]
    cur = pcd
    for n in PYRAMID:
        stride = cur.shape[2] // n
        cur = cur[:, :, ::stride][:, :, :n]
        xyzs.append(cur)
    return xyzs


def kernel(pcd, lvl0_W0, lvl0_b0, lvl0_W1, lvl0_b1, mlp0_W0, mlp0_b0, mlp0_W1, mlp0_b1, conv0_W, conv0_b, mlp1_W0, mlp1_b0, mlp1_W1, mlp1_b1, conv1_W, conv1_b, mlp2_W0, mlp2_b0, mlp2_W1, mlp2_b1, conv2_W, conv2_b):
    B, _, N0 = pcd.shape
    xyzs = _pyramid(pcd)
    Ns = [N0] + PYRAMID
    xyzT = [jnp.transpose(x, (0, 2, 1)) for x in xyzs]
    xyz8T = [jnp.pad(x, ((0, 0), (0, 0), (0, 5))) for x in xyzT]
    xyz8 = [jnp.transpose(x, (0, 2, 1)) for x in xyz8T]

    feats0T = _mlp_chain(
        xyzT[0],
        [(lvl0_W0.T, lvl0_b0[None, :]), (lvl0_W1.T, lvl0_b1[None, :])])

    mlps = [((mlp0_W0, mlp0_b0), (mlp0_W1, mlp0_b1)),
            ((mlp1_W0, mlp1_b0), (mlp1_W1, mlp1_b1)),
            ((mlp2_W0, mlp2_b0), (mlp2_W1, mlp2_b1))]
    convs = [(conv0_W, conv0_b), (conv1_W, conv1_b), (conv2_W, conv2_b)]

    TW = 128
    knn = []
    for i in range(3):
        convW, convbias = convs[i]
        Co = convW.shape[0]
        WxT = jnp.pad(jnp.transpose(convW[:, :3], (1, 0)),
                      ((0, 5), (0, 0)))
        knn.append(_knn_kernel(xyz8[i], xyz8T[i + 1], WxT,
                               convbias[None, :]))

    fT = feats0T
    featsT = [feats0T]
    for i in range(3):
        convW, convbias = convs[i]
        Co = convW.shape[0]
        layers = [(W.T, bb[None, :]) for (W, bb) in mlps[i]]
        WfT = jnp.transpose(convW[:, 3:], (1, 0))
        WxT = jnp.pad(jnp.transpose(convW[:, :3], (1, 0)),
                      ((0, 5), (0, 0)))
        WfTp = jnp.pad(WfT, ((0, 0), (0, TW - Co)))
        WxTp = jnp.pad(WxT, ((0, 0), (0, TW - Co)))
        preT = _pre_kernel(fT, xyz8T[i], layers, WfTp, WxTp)
        idxf, qT = knn[i]
        M = Ns[i + 1]
        out = _sc_gather_max(
            preT.reshape(B * Ns[i], TW),
            idxf.reshape(B * M * K_NN),
            qT.reshape(B * M, Co))
        fT = out.reshape(B, M, Co)
        featsT.append(fT)

    feats = tuple(jnp.transpose(f, (0, 2, 1)) for f in featsT)
    return feats + tuple(xyzs)
